# baseline (speedup 1.0000x reference)
import jax
import jax.numpy as jnp
from jax.experimental import pallas as pl

_BLOCK_L = 40


def _bcast_kernel(tab_ref, out_ref):
    out_ref[...] = jnp.broadcast_to(tab_ref[...][:, :, None], out_ref.shape)


def kernel(x, pos_table):
    batch = x.shape[0]
    seq_len = x.shape[1]
    emb = pos_table.shape[1]
    tab = pos_table[:seq_len]

    block_l = _BLOCK_L if seq_len % _BLOCK_L == 0 else seq_len
    grid = (seq_len // block_l,)

    out = pl.pallas_call(
        _bcast_kernel,
        grid=grid,
        in_specs=[pl.BlockSpec((block_l, emb), lambda i: (i, 0))],
        out_specs=pl.BlockSpec((block_l, emb, batch), lambda i: (i, 0, 0)),
        out_shape=jax.ShapeDtypeStruct((seq_len, emb, batch), pos_table.dtype),
    )(tab)
    return jnp.transpose(out, (2, 0, 1))


# speedup vs baseline: 1.0810x; 1.0810x over previous
import jax
import jax.numpy as jnp
from jax.experimental import pallas as pl

_BLOCK_L = 8


def _bcast_kernel(tab_ref, out_ref):
    out_ref[...] = jnp.broadcast_to(tab_ref[...][:, :, None], out_ref.shape)


def kernel(x, pos_table):
    batch = x.shape[0]
    seq_len = x.shape[1]
    emb = pos_table.shape[1]
    tab = pos_table[:seq_len]

    block_l = _BLOCK_L if seq_len % _BLOCK_L == 0 else seq_len
    grid = (seq_len // block_l,)

    out = pl.pallas_call(
        _bcast_kernel,
        grid=grid,
        in_specs=[pl.BlockSpec((block_l, emb), lambda i: (i, 0))],
        out_specs=pl.BlockSpec((block_l, emb, batch), lambda i: (i, 0, 0)),
        out_shape=jax.ShapeDtypeStruct((seq_len, emb, batch), pos_table.dtype),
    )(tab)
    return jnp.transpose(out, (2, 0, 1))
